# baseline (speedup 1.0000x reference)
import functools

import jax
import jax.numpy as jnp
from jax import lax
from jax.experimental import pallas as pl
from jax.experimental.pallas import tpu as pltpu
from jax.experimental.pallas import tpu_sc as plsc

_EMBED = 32
_LANES = 16
_GROUP = 16
_NBUF = 6

_info = plsc.get_sparse_core_info()
_NC = _info.num_cores
_NS = _info.num_subcores
_NW = _NC * _NS


def _gmf_kernel(b_per_w, ut_hbm, it_hbm, uids_hbm, iids_hbm, out_hbm,
                uids_v, iids_v, pan, obuf, sem):
    wid = lax.axis_index("s") * _NC + lax.axis_index("c")
    base = wid * b_per_w
    n_groups = b_per_w // _GROUP

    pltpu.sync_copy(uids_hbm.at[pl.ds(base, b_per_w)], uids_v)
    pltpu.sync_copy(iids_hbm.at[pl.ds(base, b_per_w)], iids_v)

    iota = lax.broadcasted_iota(jnp.int32, (_LANES,), 0)

    def run_pass(tab_hbm, ids_v, quarter, is_item):
        half16 = iota & 7
        pairsel = iota >> 3
        rows2 = half16 + quarter * 8

        def stage(jb, carry):
            lag = _NBUF - 1

            @pl.when(jb >= lag)
            def _drain():
                for g in range(_GROUP):
                    pltpu.make_async_copy(
                        tab_hbm.at[pl.ds(0, 8), pl.ds(0, 128)],
                        pan.at[lax.rem(jb - lag, _NBUF), g], sem).wait()

            @pl.when(jb < n_groups)
            def _fire():
                slot = lax.rem(jb, _NBUF)
                vec = ids_v[pl.ds(jb * _GROUP, _GROUP)]
                for g in range(_GROUP):
                    a = pl.multiple_of((vec[g] >> 7) << 7, 128)
                    pltpu.async_copy(
                        tab_hbm.at[pl.ds(quarter * 8, 8),
                                   pl.ds(a, 128)],
                        pan.at[slot, g], sem)

            @pl.when(jb >= lag)
            def _select():
                slot = lax.rem(jb - lag, _NBUF)
                jb0 = (jb - lag) * _GROUP
                vec = ids_v[pl.ds(jb0, _GROUP)]
                cvec = vec & 127
                for g in range(0, _GROUP, 2):
                    col = (jb0 + g) + pairsel
                    cval = cvec[g] * (1 - pairsel) + cvec[g + 1] * pairsel
                    pidx = g + pairsel
                    val = plsc.load_gather(pan.at[slot], [pidx, half16, cval])
                    if is_item:
                        prev = plsc.load_gather(obuf, [rows2, col])
                        val = val * prev
                    plsc.store_scatter(obuf, [rows2, col], val)

            return carry

        lax.fori_loop(0, n_groups + _NBUF - 1, stage, 0)

    for q in range(4):
        run_pass(ut_hbm, uids_v, q, False)
    for q in range(4):
        run_pass(it_hbm, iids_v, q, True)

    pltpu.sync_copy(obuf, out_hbm.at[:, pl.ds(base, b_per_w)])


def kernel(user_ids, item_ids, user_table, item_table):
    batch = user_ids.shape[0]
    b_per_w = batch // _NW
    mesh = plsc.VectorSubcoreMesh(core_axis_name="c", subcore_axis_name="s")
    run = functools.partial(
        pl.kernel,
        mesh=mesh,
        out_type=jax.ShapeDtypeStruct((_EMBED, batch), jnp.float32),
        scratch_types=[
            pltpu.VMEM((b_per_w,), jnp.int32),
            pltpu.VMEM((b_per_w,), jnp.int32),
            pltpu.VMEM((_NBUF, _GROUP, 8, 128), jnp.float32),
            pltpu.VMEM((_EMBED, b_per_w), jnp.float32),
            pltpu.SemaphoreType.DMA,
        ],
        compiler_params=pltpu.CompilerParams(needs_layout_passes=False),
    )(functools.partial(_gmf_kernel, b_per_w))
    out_t = run(user_table.T, item_table.T,
                user_ids.astype(jnp.int32), item_ids.astype(jnp.int32))
    return out_t.T


# speedup vs baseline: 1.0532x; 1.0532x over previous
import functools

import jax
import jax.numpy as jnp
from jax import lax
from jax.experimental import pallas as pl
from jax.experimental.pallas import tpu as pltpu
from jax.experimental.pallas import tpu_sc as plsc

_EMBED = 32
_LANES = 16
_GROUP = 16
_NBUF = 3

_info = plsc.get_sparse_core_info()
_NC = _info.num_cores
_NS = _info.num_subcores
_NW = _NC * _NS


def _gmf_kernel(b_per_w, ut_hbm, it_hbm, uids_hbm, iids_hbm, out_hbm,
                uids_v, iids_v, pan, obuf, sem):
    wid = lax.axis_index("s") * _NC + lax.axis_index("c")
    base = wid * b_per_w
    n_groups = b_per_w // _GROUP

    pltpu.sync_copy(uids_hbm.at[pl.ds(base, b_per_w)], uids_v)
    pltpu.sync_copy(iids_hbm.at[pl.ds(base, b_per_w)], iids_v)

    iota = lax.broadcasted_iota(jnp.int32, (_LANES,), 0)

    def run_pass(tab_hbm, ids_v, half, is_item):
        rows = iota + half * _LANES

        def stage(jb, carry):
            lag = _NBUF - 1

            @pl.when(jb >= lag)
            def _drain():
                for g in range(_GROUP):
                    pltpu.make_async_copy(
                        tab_hbm.at[pl.ds(0, _LANES), pl.ds(0, 128)],
                        pan.at[lax.rem(jb - lag, _NBUF), g], sem).wait()

            @pl.when(jb < n_groups)
            def _fire():
                slot = lax.rem(jb, _NBUF)
                vec = ids_v[pl.ds(jb * _GROUP, _GROUP)]
                for g in range(_GROUP):
                    a = pl.multiple_of((vec[g] >> 7) << 7, 128)
                    pltpu.async_copy(
                        tab_hbm.at[pl.ds(half * _LANES, _LANES),
                                   pl.ds(a, 128)],
                        pan.at[slot, g], sem)

            @pl.when(jb >= lag)
            def _select():
                slot = lax.rem(jb - lag, _NBUF)
                jb0 = (jb - lag) * _GROUP
                vec = ids_v[pl.ds(jb0, _GROUP)]
                cvec = vec & 127
                for g in range(_GROUP):
                    col = iota * 0 + (jb0 + g)
                    cval = iota * 0 + cvec[g]
                    val = plsc.load_gather(pan.at[slot, g], [iota, cval])
                    if is_item:
                        prev = plsc.load_gather(obuf, [rows, col])
                        val = val * prev
                    plsc.store_scatter(obuf, [rows, col], val)

            return carry

        lax.fori_loop(0, n_groups + _NBUF - 1, stage, 0)

    run_pass(ut_hbm, uids_v, 0, False)
    run_pass(ut_hbm, uids_v, 1, False)
    run_pass(it_hbm, iids_v, 0, True)
    run_pass(it_hbm, iids_v, 1, True)

    pltpu.sync_copy(obuf, out_hbm.at[:, pl.ds(base, b_per_w)])


def kernel(user_ids, item_ids, user_table, item_table):
    batch = user_ids.shape[0]
    b_per_w = batch // _NW
    mesh = plsc.VectorSubcoreMesh(core_axis_name="c", subcore_axis_name="s")
    run = functools.partial(
        pl.kernel,
        mesh=mesh,
        out_type=jax.ShapeDtypeStruct((_EMBED, batch), jnp.float32),
        scratch_types=[
            pltpu.VMEM((b_per_w,), jnp.int32),
            pltpu.VMEM((b_per_w,), jnp.int32),
            pltpu.VMEM((_NBUF, _GROUP, _LANES, 128), jnp.float32),
            pltpu.VMEM((_EMBED, b_per_w), jnp.float32),
            pltpu.SemaphoreType.DMA,
        ],
        compiler_params=pltpu.CompilerParams(needs_layout_passes=False),
    )(functools.partial(_gmf_kernel, b_per_w))
    out_t = run(user_table.T, item_table.T,
                user_ids.astype(jnp.int32), item_ids.astype(jnp.int32))
    return out_t.T


# speedup vs baseline: 1.0620x; 1.0084x over previous
import functools

import jax
import jax.numpy as jnp
from jax import lax
from jax.experimental import pallas as pl
from jax.experimental.pallas import tpu as pltpu
from jax.experimental.pallas import tpu_sc as plsc

_EMBED = 32
_LANES = 16
_GROUP = 16
_NBUF = 3

_info = plsc.get_sparse_core_info()
_NC = _info.num_cores
_NS = _info.num_subcores
_NW = _NC * _NS


def _gmf_kernel(b_per_w, ut_hbm, it_hbm, uids_hbm, iids_hbm, out_hbm,
                uids_v, iids_v, pan, obuf, sem):
    wid = lax.axis_index("s") * _NC + lax.axis_index("c")
    base = wid * b_per_w
    n_groups = b_per_w // _GROUP

    pltpu.sync_copy(uids_hbm.at[pl.ds(base, b_per_w)], uids_v)
    pltpu.sync_copy(iids_hbm.at[pl.ds(base, b_per_w)], iids_v)

    iota = lax.broadcasted_iota(jnp.int32, (_LANES,), 0)

    def run_pass(tab_hbm, ids_v, half, is_item):
        rows = iota + half * _LANES

        def stage(jb, carry):
            lag = _NBUF - 1

            @pl.when(jb >= lag)
            def _drain():
                for g in range(_GROUP):
                    pltpu.make_async_copy(
                        tab_hbm.at[pl.ds(0, _LANES), pl.ds(0, 128)],
                        pan.at[lax.rem(jb - lag, _NBUF), g], sem).wait()

            @pl.when(jb < n_groups)
            def _fire():
                slot = lax.rem(jb, _NBUF)
                vec = ids_v[pl.ds(jb * _GROUP, _GROUP)]
                for g in range(_GROUP):
                    a = pl.multiple_of((vec[g] >> 7) << 7, 128)
                    pltpu.async_copy(
                        tab_hbm.at[pl.ds(half * _LANES, _LANES),
                                   pl.ds(a, 128)],
                        pan.at[slot, g], sem)

            @pl.when(jb >= lag)
            def _select():
                slot = lax.rem(jb - lag, _NBUF)
                jb0 = (jb - lag) * _GROUP
                vec = ids_v[pl.ds(jb0, _GROUP)]
                cvec = vec & 127
                for d in range(_LANES):
                    dvec = iota * 0 + d
                    val = plsc.load_gather(pan.at[slot], [iota, dvec, cvec])
                    row = half * _LANES + d
                    if is_item:
                        val = val * obuf[row, pl.ds(jb0, _GROUP)]
                    obuf[row, pl.ds(jb0, _GROUP)] = val

            return carry

        lax.fori_loop(0, n_groups + _NBUF - 1, stage, 0)

    run_pass(ut_hbm, uids_v, 0, False)
    run_pass(ut_hbm, uids_v, 1, False)
    run_pass(it_hbm, iids_v, 0, True)
    run_pass(it_hbm, iids_v, 1, True)

    pltpu.sync_copy(obuf, out_hbm.at[:, pl.ds(base, b_per_w)])


def kernel(user_ids, item_ids, user_table, item_table):
    batch = user_ids.shape[0]
    b_per_w = batch // _NW
    mesh = plsc.VectorSubcoreMesh(core_axis_name="c", subcore_axis_name="s")
    run = functools.partial(
        pl.kernel,
        mesh=mesh,
        out_type=jax.ShapeDtypeStruct((_EMBED, batch), jnp.float32),
        scratch_types=[
            pltpu.VMEM((b_per_w,), jnp.int32),
            pltpu.VMEM((b_per_w,), jnp.int32),
            pltpu.VMEM((_NBUF, _GROUP, _LANES, 128), jnp.float32),
            pltpu.VMEM((_EMBED, b_per_w), jnp.float32),
            pltpu.SemaphoreType.DMA,
        ],
        compiler_params=pltpu.CompilerParams(needs_layout_passes=False),
    )(functools.partial(_gmf_kernel, b_per_w))
    out_t = run(user_table.T, item_table.T,
                user_ids.astype(jnp.int32), item_ids.astype(jnp.int32))
    return out_t.T


# speedup vs baseline: 1.0633x; 1.0012x over previous
import functools

import jax
import jax.numpy as jnp
from jax import lax
from jax.experimental import pallas as pl
from jax.experimental.pallas import tpu as pltpu
from jax.experimental.pallas import tpu_sc as plsc

_EMBED = 32
_LANES = 16
_GROUP = 16
_NBUF = 3

_info = plsc.get_sparse_core_info()
_NC = _info.num_cores
_NS = _info.num_subcores
_NW = _NC * _NS


def _gmf_kernel(b_per_w, ut_hbm, it_hbm, uids_hbm, iids_hbm, out_hbm,
                uids_v, iids_v, pan, obuf, sem):
    wid = lax.axis_index("s") * _NC + lax.axis_index("c")
    base = wid * b_per_w
    n_groups = b_per_w // _GROUP

    pltpu.sync_copy(uids_hbm.at[pl.ds(base, b_per_w)], uids_v)
    pltpu.sync_copy(iids_hbm.at[pl.ds(base, b_per_w)], iids_v)

    iota = lax.broadcasted_iota(jnp.int32, (_LANES,), 0)

    def run_pass(tab_hbm, ids_v, half, is_item):
        def stage(jb, carry):
            lag = _NBUF - 1

            @pl.when(jb >= lag)
            def _drain():
                for g in range(_GROUP):
                    pltpu.make_async_copy(
                        tab_hbm.at[pl.ds(0, _LANES), pl.ds(0, 128)],
                        pan.at[lax.rem(jb - lag, _NBUF), g], sem).wait()

            @pl.when(jb < n_groups)
            def _fire():
                slot = lax.rem(jb, _NBUF)
                vec = ids_v[pl.ds(jb * _GROUP, _GROUP)]
                for g in range(_GROUP):
                    a = pl.multiple_of((vec[g] >> 7) << 7, 128)
                    pltpu.async_copy(
                        tab_hbm.at[pl.ds(half * _LANES, _LANES),
                                   pl.ds(a, 128)],
                        pan.at[slot, g], sem)

            @pl.when(jb >= lag)
            def _select():
                slot = lax.rem(jb - lag, _NBUF)
                jb0 = (jb - lag) * _GROUP
                vec = ids_v[pl.ds(jb0, _GROUP)]
                cvec = vec & 127
                for d in range(_LANES):
                    dvec = iota * 0 + d
                    val = plsc.load_gather(pan.at[slot], [iota, dvec, cvec])
                    row = half * _LANES + d
                    if is_item:
                        val = val * obuf[row, pl.ds(jb0, _GROUP)]
                    obuf[row, pl.ds(jb0, _GROUP)] = val

            return carry

        lax.fori_loop(0, n_groups + _NBUF - 1, stage, 0)

    run_pass(ut_hbm, uids_v, 0, False)
    run_pass(ut_hbm, uids_v, 1, False)
    run_pass(it_hbm, iids_v, 0, True)
    run_pass(it_hbm, iids_v, 1, True)

    pltpu.sync_copy(obuf, out_hbm.at[:, pl.ds(base, b_per_w)])


def kernel(user_ids, item_ids, user_table, item_table):
    batch = user_ids.shape[0]
    b_per_w = batch // _NW
    mesh = plsc.VectorSubcoreMesh(core_axis_name="c", subcore_axis_name="s")
    run = functools.partial(
        pl.kernel,
        mesh=mesh,
        out_type=jax.ShapeDtypeStruct((_EMBED, batch), jnp.float32),
        scratch_types=[
            pltpu.VMEM((b_per_w,), jnp.int32),
            pltpu.VMEM((b_per_w,), jnp.int32),
            pltpu.VMEM((_NBUF, _GROUP, _LANES, 128), jnp.float32),
            pltpu.VMEM((_EMBED, b_per_w), jnp.float32),
            pltpu.SemaphoreType.DMA,
        ],
        compiler_params=pltpu.CompilerParams(needs_layout_passes=False),
    )(functools.partial(_gmf_kernel, b_per_w))
    out_t = run(user_table.T, item_table.T,
                user_ids.astype(jnp.int32), item_ids.astype(jnp.int32))
    return out_t.T
